# Initial kernel scaffold; baseline (speedup 1.0000x reference)
import jax
import jax.numpy as jnp
from jax.experimental import pallas as pl


def kernel(x, edge_index, W1, b1, W2, b2):
    raise NotImplementedError("write your pallas kernel here")


# speedup vs baseline: 49.2359x; 49.2359x over previous
import functools

import jax
import jax.numpy as jnp
from jax import lax
from jax.experimental import pallas as pl
from jax.experimental.pallas import tpu as pltpu
from jax.experimental.pallas import tpu_sc as plsc

N = 10000
D_IN = 128
D_HID = 64
E = 320000
NC = 2
NS = 16
NW = NC * NS
EPW = E // NW
CH = 125
K = EPW // CH
NPAD = 10240
RT = NPAD // NS

_mesh = plsc.VectorSubcoreMesh(core_axis_name="c", subcore_axis_name="s")


@functools.partial(
    pl.kernel,
    out_type=jax.ShapeDtypeStruct((NC, NPAD), jnp.float32),
    mesh=_mesh,
    compiler_params=pltpu.CompilerParams(use_tc_tiling_on_sc=False),
    scratch_types=[
        pltpu.VMEM((K, CH), jnp.int32),
        pltpu.VMEM((128,), jnp.float32),
        pltpu.VMEM_SHARED((NPAD,), jnp.float32),
    ],
)
def _sc_degree(dst_hbm, ones_hbm, zeros_hbm, out_hbm, didx, ones_v, acc):
    cid = lax.axis_index("c")
    sid = lax.axis_index("s")
    wid = sid * NC + cid
    pltpu.sync_copy(dst_hbm.at[wid], didx)
    pltpu.sync_copy(ones_hbm, ones_v)
    pltpu.sync_copy(zeros_hbm.at[pl.ds(sid * RT, RT)], acc.at[pl.ds(sid * RT, RT)])
    plsc.subcore_barrier()

    def step(j, carry):
        pltpu.sync_copy(ones_v.at[pl.ds(0, CH)], acc.at[didx.at[j]], add=True)
        return carry

    lax.fori_loop(0, K, step, 0)
    plsc.subcore_barrier()
    pltpu.sync_copy(acc.at[pl.ds(sid * RT, RT)], out_hbm.at[cid, pl.ds(sid * RT, RT)])


@functools.partial(
    pl.kernel,
    out_type=jax.ShapeDtypeStruct((NC, NPAD, D_HID), jnp.float32),
    mesh=_mesh,
    compiler_params=pltpu.CompilerParams(use_tc_tiling_on_sc=False),
    scratch_types=[
        pltpu.VMEM((K, CH), jnp.int32),
        pltpu.VMEM((K, CH), jnp.int32),
        pltpu.VMEM((CH, D_HID), jnp.float32),
        pltpu.VMEM((CH, D_HID), jnp.float32),
        pltpu.SemaphoreType.DMA,
        pltpu.SemaphoreType.DMA,
        pltpu.VMEM_SHARED((NPAD, D_HID), jnp.float32),
    ],
)
def _sc_agg64(g1_hbm, src_hbm, dst_hbm, zeros_hbm, out_hbm,
              sidx, didx, rows0, rows1, sem0, sem1, acc):
    cid = lax.axis_index("c")
    sid = lax.axis_index("s")
    wid = sid * NC + cid
    pltpu.sync_copy(src_hbm.at[wid], sidx)
    pltpu.sync_copy(dst_hbm.at[wid], didx)
    pltpu.sync_copy(zeros_hbm.at[pl.ds(sid * RT, RT)], acc.at[pl.ds(sid * RT, RT)])
    plsc.subcore_barrier()

    def step(jj, carry):
        j0 = 2 * jj
        d0 = pltpu.async_copy(g1_hbm.at[sidx.at[j0]], rows0, sem0)
        d1 = pltpu.async_copy(g1_hbm.at[sidx.at[j0 + 1]], rows1, sem1)
        d0.wait()
        pltpu.sync_copy(rows0, acc.at[didx.at[j0]], add=True)
        d1.wait()
        pltpu.sync_copy(rows1, acc.at[didx.at[j0 + 1]], add=True)
        return carry

    lax.fori_loop(0, K // 2, step, 0)
    plsc.subcore_barrier()
    pltpu.sync_copy(acc.at[pl.ds(sid * RT, RT)], out_hbm.at[cid, pl.ds(sid * RT, RT)])


@functools.partial(
    pl.kernel,
    out_type=jax.ShapeDtypeStruct((NC, NPAD), jnp.float32),
    mesh=_mesh,
    compiler_params=pltpu.CompilerParams(use_tc_tiling_on_sc=False),
    scratch_types=[
        pltpu.VMEM((K, CH), jnp.int32),
        pltpu.VMEM((K, CH), jnp.int32),
        pltpu.VMEM((CH,), jnp.float32),
        pltpu.VMEM((CH,), jnp.float32),
        pltpu.SemaphoreType.DMA,
        pltpu.SemaphoreType.DMA,
        pltpu.VMEM_SHARED((NPAD,), jnp.float32),
        pltpu.VMEM_SHARED((N,), jnp.float32),
    ],
)
def _sc_agg1(u_hbm, src_hbm, dst_hbm, zeros_hbm, out_hbm,
             sidx, didx, vals0, vals1, sem0, sem1, acc, us):
    cid = lax.axis_index("c")
    sid = lax.axis_index("s")
    wid = sid * NC + cid
    pltpu.sync_copy(src_hbm.at[wid], sidx)
    pltpu.sync_copy(dst_hbm.at[wid], didx)
    pltpu.sync_copy(zeros_hbm.at[pl.ds(sid * RT, RT)], acc.at[pl.ds(sid * RT, RT)])

    @pl.when(sid == 0)
    def _stage():
        pltpu.sync_copy(u_hbm, us)

    plsc.subcore_barrier()

    def step(jj, carry):
        j0 = 2 * jj
        d0 = pltpu.async_copy(us.at[sidx.at[j0]], vals0, sem0)
        d1 = pltpu.async_copy(us.at[sidx.at[j0 + 1]], vals1, sem1)
        d0.wait()
        pltpu.sync_copy(vals0, acc.at[didx.at[j0]], add=True)
        d1.wait()
        pltpu.sync_copy(vals1, acc.at[didx.at[j0 + 1]], add=True)
        return carry

    lax.fori_loop(0, K // 2, step, 0)
    plsc.subcore_barrier()
    pltpu.sync_copy(acc.at[pl.ds(sid * RT, RT)], out_hbm.at[cid, pl.ds(sid * RT, RT)])


_BM = 2000


def _tc1_body(deg_ref, x_ref, w1_ref, g1_ref, dinv_ref):
    deg = deg_ref[0, 0] + deg_ref[0, 1] + 1.0
    dinv = lax.rsqrt(deg)
    h = jnp.dot(x_ref[...], w1_ref[...], preferred_element_type=jnp.float32)
    g1_ref[...] = h * dinv[:, None]
    dinv_ref[...] = dinv[:, None]


def _tc2_body(s_ref, g1_ref, dinv_ref, w2_ref, b1_ref, u_ref):
    s = s_ref[0] + s_ref[1] + g1_ref[...]
    h = jnp.maximum(s * dinv_ref[...] + b1_ref[...], 0.0)
    u_ref[...] = jnp.dot(h, w2_ref[...], preferred_element_type=jnp.float32) * dinv_ref[...]


def _tc3_body(s2_ref, u_ref, dinv_ref, b2_ref, o_ref):
    s2 = s2_ref[0, 0] + s2_ref[0, 1]
    z = dinv_ref[...] * (s2[:, None] + u_ref[...]) + b2_ref[...]
    o_ref[...] = jax.nn.sigmoid(z)


def kernel(x, edge_index, W1, b1, W2, b2):
    ei = edge_index.astype(jnp.int32)
    src3 = ei[0].reshape(NW, K, CH)
    dst3 = ei[1].reshape(NW, K, CH)
    zeros64 = jnp.zeros((NPAD, D_HID), jnp.float32)
    zeros1 = jnp.zeros((NPAD,), jnp.float32)
    ones128 = jnp.ones((128,), jnp.float32)

    deg2 = _sc_degree(dst3, ones128, zeros1)

    grid = N // _BM
    deg2r = deg2[:, :N].reshape(NC, grid, _BM).transpose(1, 0, 2)
    g1, dinv = pl.pallas_call(
        _tc1_body,
        grid=(grid,),
        in_specs=[
            pl.BlockSpec((1, NC, _BM), lambda i: (i, 0, 0)),
            pl.BlockSpec((_BM, D_IN), lambda i: (i, 0)),
            pl.BlockSpec((D_IN, D_HID), lambda i: (0, 0)),
        ],
        out_specs=[
            pl.BlockSpec((_BM, D_HID), lambda i: (i, 0)),
            pl.BlockSpec((_BM, 1), lambda i: (i, 0)),
        ],
        out_shape=[
            jax.ShapeDtypeStruct((NPAD, D_HID), jnp.float32),
            jax.ShapeDtypeStruct((N, 1), jnp.float32),
        ],
    )(deg2r, x, W1)

    s64 = _sc_agg64(g1, src3, dst3, zeros64)

    u = pl.pallas_call(
        _tc2_body,
        grid=(grid,),
        in_specs=[
            pl.BlockSpec((NC, _BM, D_HID), lambda i: (0, i, 0)),
            pl.BlockSpec((_BM, D_HID), lambda i: (i, 0)),
            pl.BlockSpec((_BM, 1), lambda i: (i, 0)),
            pl.BlockSpec((D_HID, 1), lambda i: (0, 0)),
            pl.BlockSpec((1, D_HID), lambda i: (0, 0)),
        ],
        out_specs=pl.BlockSpec((_BM, 1), lambda i: (i, 0)),
        out_shape=jax.ShapeDtypeStruct((N, 1), jnp.float32),
    )(s64, g1, dinv, W2, b1.reshape(1, D_HID))

    s2 = _sc_agg1(u.reshape(N), src3, dst3, zeros1)

    s2r = s2[:, :N].reshape(NC, grid, _BM).transpose(1, 0, 2)
    out = pl.pallas_call(
        _tc3_body,
        grid=(grid,),
        in_specs=[
            pl.BlockSpec((1, NC, _BM), lambda i: (i, 0, 0)),
            pl.BlockSpec((_BM, 1), lambda i: (i, 0)),
            pl.BlockSpec((_BM, 1), lambda i: (i, 0)),
            pl.BlockSpec((1, 1), lambda i: (0, 0)),
        ],
        out_specs=pl.BlockSpec((_BM, 1), lambda i: (i, 0)),
        out_shape=jax.ShapeDtypeStruct((N, 1), jnp.float32),
    )(s2r, u, dinv, b2.reshape(1, 1))

    return out


# speedup vs baseline: 62.8684x; 1.2769x over previous
import functools

import jax
import jax.numpy as jnp
from jax import lax
from jax.experimental import pallas as pl
from jax.experimental.pallas import tpu as pltpu
from jax.experimental.pallas import tpu_sc as plsc

N = 10000
D_IN = 128
D_HID = 64
E = 320000
NC = 2
NS = 16
NW = NC * NS
EPW = E // NW
CH = 125
K = EPW // CH
NBUF = 4
NPAD = 10240
RT = NPAD // NS

_mesh = plsc.VectorSubcoreMesh(core_axis_name="c", subcore_axis_name="s")
_sc_params = pltpu.CompilerParams(use_tc_tiling_on_sc=False)


@functools.partial(
    pl.kernel,
    out_type=jax.ShapeDtypeStruct((NC, NPAD), jnp.float32),
    mesh=_mesh,
    compiler_params=_sc_params,
    scratch_types=[
        pltpu.VMEM((K, CH), jnp.int32),
        pltpu.VMEM((128,), jnp.float32),
        pltpu.SemaphoreType.DMA,
        pltpu.SemaphoreType.DMA,
        pltpu.VMEM_SHARED((NPAD,), jnp.float32),
    ],
)
def _sc_degree(dst_hbm, ones_hbm, zeros_hbm, out_hbm, didx, ones_v, s0, s1, acc):
    cid = lax.axis_index("c")
    sid = lax.axis_index("s")
    wid = sid * NC + cid
    pltpu.sync_copy(dst_hbm.at[wid], didx)
    pltpu.sync_copy(ones_hbm, ones_v)
    pltpu.sync_copy(zeros_hbm.at[pl.ds(sid * RT, RT)], acc.at[pl.ds(sid * RT, RT)])
    plsc.subcore_barrier()

    ones_row = ones_v.at[pl.ds(0, CH)]
    sems = [s0, s1]

    def step(jj, carry):
        j0 = 2 * jj
        pltpu.async_copy(ones_row, acc.at[didx.at[j0]], s0, add=True)
        pltpu.async_copy(ones_row, acc.at[didx.at[j0 + 1]], s1, add=True)
        pltpu.make_async_copy(ones_row, acc.at[didx.at[j0]], s0).wait()
        pltpu.make_async_copy(ones_row, acc.at[didx.at[j0 + 1]], s1).wait()
        return carry

    lax.fori_loop(0, K // 2, step, 0)
    plsc.subcore_barrier()
    pltpu.sync_copy(acc.at[pl.ds(sid * RT, RT)], out_hbm.at[cid, pl.ds(sid * RT, RT)])


@functools.partial(
    pl.kernel,
    out_type=jax.ShapeDtypeStruct((NC, NPAD, D_HID), jnp.float32),
    mesh=_mesh,
    compiler_params=_sc_params,
    scratch_types=(
        [pltpu.VMEM((K, CH), jnp.int32)] * 2
        + [pltpu.VMEM((CH, D_HID), jnp.float32)] * NBUF
        + [pltpu.SemaphoreType.DMA] * (2 * NBUF)
        + [pltpu.VMEM_SHARED((NPAD, D_HID), jnp.float32)]
    ),
)
def _sc_agg64(g1_hbm, src_hbm, dst_hbm, zeros_hbm, out_hbm, sidx, didx, *rest):
    rows = rest[:NBUF]
    gsem = rest[NBUF:2 * NBUF]
    ssem = rest[2 * NBUF:3 * NBUF]
    acc = rest[3 * NBUF]
    cid = lax.axis_index("c")
    sid = lax.axis_index("s")
    wid = sid * NC + cid
    pltpu.sync_copy(src_hbm.at[wid], sidx)
    pltpu.sync_copy(dst_hbm.at[wid], didx)
    pltpu.sync_copy(zeros_hbm.at[pl.ds(sid * RT, RT)], acc.at[pl.ds(sid * RT, RT)])
    plsc.subcore_barrier()

    for b in range(NBUF):
        pltpu.async_copy(g1_hbm.at[sidx.at[b]], rows[b], gsem[b])

    def step(jj, carry):
        for b in range(NBUF):
            j = NBUF * jj + b
            pltpu.make_async_copy(g1_hbm.at[sidx.at[j]], rows[b], gsem[b]).wait()
            pltpu.async_copy(rows[b], acc.at[didx.at[j]], ssem[b], add=True)

            @pl.when(j + NBUF < K)
            def _refill():
                pltpu.make_async_copy(rows[b], acc.at[didx.at[j]], ssem[b]).wait()
                pltpu.async_copy(g1_hbm.at[sidx.at[j + NBUF]], rows[b], gsem[b])

        return carry

    lax.fori_loop(0, K // NBUF, step, 0)
    for b in range(NBUF):
        j = K - NBUF + b
        pltpu.make_async_copy(rows[b], acc.at[didx.at[j]], ssem[b]).wait()
    plsc.subcore_barrier()
    pltpu.sync_copy(acc.at[pl.ds(sid * RT, RT)], out_hbm.at[cid, pl.ds(sid * RT, RT)])


@functools.partial(
    pl.kernel,
    out_type=jax.ShapeDtypeStruct((NC, NPAD), jnp.float32),
    mesh=_mesh,
    compiler_params=_sc_params,
    scratch_types=(
        [pltpu.VMEM((K, CH), jnp.int32)] * 2
        + [pltpu.VMEM((CH,), jnp.float32)] * NBUF
        + [pltpu.SemaphoreType.DMA] * (2 * NBUF)
        + [pltpu.VMEM_SHARED((NPAD,), jnp.float32),
           pltpu.VMEM_SHARED((N,), jnp.float32)]
    ),
)
def _sc_agg1(u_hbm, src_hbm, dst_hbm, zeros_hbm, out_hbm, sidx, didx, *rest):
    vals = rest[:NBUF]
    gsem = rest[NBUF:2 * NBUF]
    ssem = rest[2 * NBUF:3 * NBUF]
    acc = rest[3 * NBUF]
    us = rest[3 * NBUF + 1]
    cid = lax.axis_index("c")
    sid = lax.axis_index("s")
    wid = sid * NC + cid
    pltpu.sync_copy(src_hbm.at[wid], sidx)
    pltpu.sync_copy(dst_hbm.at[wid], didx)
    pltpu.sync_copy(zeros_hbm.at[pl.ds(sid * RT, RT)], acc.at[pl.ds(sid * RT, RT)])

    @pl.when(sid == 0)
    def _stage():
        pltpu.sync_copy(u_hbm, us)

    plsc.subcore_barrier()

    for b in range(NBUF):
        pltpu.async_copy(us.at[sidx.at[b]], vals[b], gsem[b])

    def step(jj, carry):
        for b in range(NBUF):
            j = NBUF * jj + b
            pltpu.make_async_copy(us.at[sidx.at[j]], vals[b], gsem[b]).wait()
            pltpu.async_copy(vals[b], acc.at[didx.at[j]], ssem[b], add=True)

            @pl.when(j + NBUF < K)
            def _refill():
                pltpu.make_async_copy(vals[b], acc.at[didx.at[j]], ssem[b]).wait()
                pltpu.async_copy(us.at[sidx.at[j + NBUF]], vals[b], gsem[b])

        return carry

    lax.fori_loop(0, K // NBUF, step, 0)
    for b in range(NBUF):
        j = K - NBUF + b
        pltpu.make_async_copy(vals[b], acc.at[didx.at[j]], ssem[b]).wait()
    plsc.subcore_barrier()
    pltpu.sync_copy(acc.at[pl.ds(sid * RT, RT)], out_hbm.at[cid, pl.ds(sid * RT, RT)])


def _tc1_body(deg_ref, x_ref, w1_ref, g1_ref, dinv_ref):
    deg = deg_ref[0] + deg_ref[1] + 1.0
    dinv = lax.rsqrt(deg)
    h = jnp.dot(x_ref[...], w1_ref[...], preferred_element_type=jnp.float32)
    dinv_n = dinv[:N]
    g1_ref[pl.ds(0, N), :] = h * dinv_n[:, None]
    dinv_ref[...] = dinv_n[:, None]


def _tc2_body(s_ref, g1_ref, dinv_ref, w2_ref, b1_ref, u_ref):
    s = s_ref[0, pl.ds(0, N), :] + s_ref[1, pl.ds(0, N), :] + g1_ref[pl.ds(0, N), :]
    h = jnp.maximum(s * dinv_ref[...] + b1_ref[...], 0.0)
    u_ref[...] = jnp.dot(h, w2_ref[...], preferred_element_type=jnp.float32) * dinv_ref[...]


def _tc3_body(s2_ref, u_ref, dinv_ref, b2_ref, o_ref):
    s2 = s2_ref[0] + s2_ref[1]
    z = dinv_ref[...] * (s2[:N, None] + u_ref[...]) + b2_ref[...]
    o_ref[...] = jax.nn.sigmoid(z)


def kernel(x, edge_index, W1, b1, W2, b2):
    ei = edge_index.astype(jnp.int32)
    src3 = ei[0].reshape(NW, K, CH)
    dst3 = ei[1].reshape(NW, K, CH)
    zeros64 = jnp.zeros((NPAD, D_HID), jnp.float32)
    zeros1 = jnp.zeros((NPAD,), jnp.float32)
    ones128 = jnp.ones((128,), jnp.float32)

    deg2 = _sc_degree(dst3, ones128, zeros1)

    g1, dinv = pl.pallas_call(
        _tc1_body,
        out_shape=[
            jax.ShapeDtypeStruct((NPAD, D_HID), jnp.float32),
            jax.ShapeDtypeStruct((N, 1), jnp.float32),
        ],
    )(deg2, x, W1)

    s64 = _sc_agg64(g1, src3, dst3, zeros64)

    u = pl.pallas_call(
        _tc2_body,
        out_shape=jax.ShapeDtypeStruct((N, 1), jnp.float32),
    )(s64, g1, dinv, W2, b1.reshape(1, D_HID))

    s2 = _sc_agg1(u.reshape(N), src3, dst3, zeros1)

    out = pl.pallas_call(
        _tc3_body,
        out_shape=jax.ShapeDtypeStruct((N, 1), jnp.float32),
    )(s2, u, dinv, b2.reshape(1, 1))

    return out


# speedup vs baseline: 70.4216x; 1.1201x over previous
import functools

import jax
import jax.numpy as jnp
from jax import lax
from jax.experimental import pallas as pl
from jax.experimental.pallas import tpu as pltpu
from jax.experimental.pallas import tpu_sc as plsc

N = 10000
D_IN = 128
D_HID = 64
E = 320000
NC = 2
NS = 16
NW = NC * NS
EPW = E // NW
CH = 125
K = EPW // CH
NBUF = 4
NPAD = 10240
RT = NPAD // NS
NF = NPAD // 128
_BM = 2048
_GRID = NPAD // _BM
_BF = NF // _GRID

_mesh = plsc.VectorSubcoreMesh(core_axis_name="c", subcore_axis_name="s")
_sc_params = pltpu.CompilerParams(
    use_tc_tiling_on_sc=False, disable_bounds_checks=True)


@functools.partial(
    pl.kernel,
    out_type=jax.ShapeDtypeStruct((NC, NPAD), jnp.float32),
    mesh=_mesh,
    compiler_params=_sc_params,
    scratch_types=[
        pltpu.VMEM((K, CH), jnp.int32),
        pltpu.VMEM((128,), jnp.float32),
        pltpu.SemaphoreType.DMA,
        pltpu.SemaphoreType.DMA,
        pltpu.VMEM_SHARED((NPAD,), jnp.float32),
    ],
)
def _sc_degree(dst_hbm, ones_hbm, zeros_hbm, out_hbm, didx, ones_v, s0, s1, acc):
    cid = lax.axis_index("c")
    sid = lax.axis_index("s")
    wid = sid * NC + cid
    pltpu.sync_copy(dst_hbm.at[wid], didx)
    pltpu.sync_copy(ones_hbm, ones_v)
    pltpu.sync_copy(zeros_hbm.at[pl.ds(sid * RT, RT)], acc.at[pl.ds(sid * RT, RT)])
    plsc.subcore_barrier()

    ones_row = ones_v.at[pl.ds(0, CH)]

    def step(jj, carry):
        j0 = 2 * jj
        pltpu.async_copy(ones_row, acc.at[didx.at[j0]], s0, add=True)
        pltpu.async_copy(ones_row, acc.at[didx.at[j0 + 1]], s1, add=True)
        pltpu.make_async_copy(ones_row, acc.at[didx.at[j0]], s0).wait()
        pltpu.make_async_copy(ones_row, acc.at[didx.at[j0 + 1]], s1).wait()
        return carry

    lax.fori_loop(0, K // 2, step, 0)
    plsc.subcore_barrier()
    pltpu.sync_copy(acc.at[pl.ds(sid * RT, RT)], out_hbm.at[cid, pl.ds(sid * RT, RT)])


@functools.partial(
    pl.kernel,
    out_type=jax.ShapeDtypeStruct((NC, NPAD, D_HID), jnp.float32),
    mesh=_mesh,
    compiler_params=_sc_params,
    scratch_types=(
        [pltpu.VMEM((K, CH), jnp.int32)] * 2
        + [pltpu.VMEM((CH, D_HID), jnp.float32)] * NBUF
        + [pltpu.SemaphoreType.DMA] * (2 * NBUF)
        + [pltpu.VMEM_SHARED((NPAD, D_HID), jnp.float32)]
    ),
)
def _sc_agg64(g1_hbm, src_hbm, dst_hbm, zeros_hbm, out_hbm, sidx, didx, *rest):
    rows = rest[:NBUF]
    gsem = rest[NBUF:2 * NBUF]
    ssem = rest[2 * NBUF:3 * NBUF]
    acc = rest[3 * NBUF]
    cid = lax.axis_index("c")
    sid = lax.axis_index("s")
    wid = sid * NC + cid
    pltpu.sync_copy(src_hbm.at[wid], sidx)
    pltpu.sync_copy(dst_hbm.at[wid], didx)
    pltpu.sync_copy(zeros_hbm.at[pl.ds(sid * RT, RT)], acc.at[pl.ds(sid * RT, RT)])
    plsc.subcore_barrier()

    for b in range(NBUF):
        pltpu.async_copy(g1_hbm.at[sidx.at[b]], rows[b], gsem[b])

    def step(jj, carry):
        for b in range(NBUF):
            j = NBUF * jj + b
            pltpu.make_async_copy(g1_hbm.at[sidx.at[j]], rows[b], gsem[b]).wait()
            pltpu.async_copy(rows[b], acc.at[didx.at[j]], ssem[b], add=True)

            @pl.when(j + NBUF < K)
            def _refill():
                pltpu.make_async_copy(rows[b], acc.at[didx.at[j]], ssem[b]).wait()
                pltpu.async_copy(g1_hbm.at[sidx.at[j + NBUF]], rows[b], gsem[b])

        return carry

    lax.fori_loop(0, K // NBUF, step, 0)
    for b in range(NBUF):
        j = K - NBUF + b
        pltpu.make_async_copy(rows[b], acc.at[didx.at[j]], ssem[b]).wait()
    plsc.subcore_barrier()
    pltpu.sync_copy(acc.at[pl.ds(sid * RT, RT)], out_hbm.at[cid, pl.ds(sid * RT, RT)])


@functools.partial(
    pl.kernel,
    out_type=jax.ShapeDtypeStruct((NC, NPAD), jnp.float32),
    mesh=_mesh,
    compiler_params=_sc_params,
    scratch_types=(
        [pltpu.VMEM((K, CH), jnp.int32)] * 2
        + [pltpu.VMEM((CH,), jnp.float32)] * NBUF
        + [pltpu.SemaphoreType.DMA] * (2 * NBUF)
        + [pltpu.VMEM_SHARED((NPAD,), jnp.float32),
           pltpu.VMEM_SHARED((NPAD,), jnp.float32)]
    ),
)
def _sc_agg1(u_hbm, src_hbm, dst_hbm, zeros_hbm, out_hbm, sidx, didx, *rest):
    vals = rest[:NBUF]
    gsem = rest[NBUF:2 * NBUF]
    ssem = rest[2 * NBUF:3 * NBUF]
    acc = rest[3 * NBUF]
    us = rest[3 * NBUF + 1]
    cid = lax.axis_index("c")
    sid = lax.axis_index("s")
    wid = sid * NC + cid
    pltpu.sync_copy(src_hbm.at[wid], sidx)
    pltpu.sync_copy(dst_hbm.at[wid], didx)
    pltpu.sync_copy(zeros_hbm.at[pl.ds(sid * RT, RT)], acc.at[pl.ds(sid * RT, RT)])
    pltpu.sync_copy(u_hbm.at[pl.ds(sid * RT, RT)], us.at[pl.ds(sid * RT, RT)])
    plsc.subcore_barrier()

    for b in range(NBUF):
        pltpu.async_copy(us.at[sidx.at[b]], vals[b], gsem[b])

    def step(jj, carry):
        for b in range(NBUF):
            j = NBUF * jj + b
            pltpu.make_async_copy(us.at[sidx.at[j]], vals[b], gsem[b]).wait()
            pltpu.async_copy(vals[b], acc.at[didx.at[j]], ssem[b], add=True)

            @pl.when(j + NBUF < K)
            def _refill():
                pltpu.make_async_copy(vals[b], acc.at[didx.at[j]], ssem[b]).wait()
                pltpu.async_copy(us.at[sidx.at[j + NBUF]], vals[b], gsem[b])

        return carry

    lax.fori_loop(0, K // NBUF, step, 0)
    for b in range(NBUF):
        j = K - NBUF + b
        pltpu.make_async_copy(vals[b], acc.at[didx.at[j]], ssem[b]).wait()
    plsc.subcore_barrier()
    pltpu.sync_copy(acc.at[pl.ds(sid * RT, RT)], out_hbm.at[cid, pl.ds(sid * RT, RT)])


def _tc1_body(deg_ref, x_ref, w1_ref, g1_ref):
    dinv_c = lax.rsqrt(deg_ref[0] + deg_ref[1] + 1.0)[:, None]
    h = jnp.dot(x_ref[...], w1_ref[...], preferred_element_type=jnp.float32)
    g1_ref[...] = h * dinv_c


def _tc2_body(deg_ref, s_ref, g1_ref, w2_ref, b1_ref, u_ref):
    dinv_c = lax.rsqrt(deg_ref[0] + deg_ref[1] + 1.0)[:, None]
    s = s_ref[0] + s_ref[1] + g1_ref[...]
    h = jnp.maximum(s * dinv_c + b1_ref[...], 0.0)
    u_col = jnp.dot(h, w2_ref[...], preferred_element_type=jnp.float32) * dinv_c
    u_ref[...] = u_col.reshape(_BM).reshape(_BF, 128)


def _tc3_body(deg_ref, s2_ref, u_ref, b2_ref, o_ref):
    dinv = lax.rsqrt(deg_ref[0] + deg_ref[1] + 1.0)
    z = dinv * (s2_ref[0] + s2_ref[1] + u_ref[...]) + b2_ref[...]
    o_ref[...] = jax.nn.sigmoid(z)


def kernel(x, edge_index, W1, b1, W2, b2):
    ei = edge_index.astype(jnp.int32)
    src3 = ei[0].reshape(NW, K, CH)
    dst3 = ei[1].reshape(NW, K, CH)
    zeros64 = jnp.zeros((NPAD, D_HID), jnp.float32)
    zeros1 = jnp.zeros((NPAD,), jnp.float32)
    ones128 = jnp.ones((128,), jnp.float32)

    deg2 = _sc_degree(dst3, ones128, zeros1)

    g1 = pl.pallas_call(
        _tc1_body,
        grid=(_GRID,),
        in_specs=[
            pl.BlockSpec((NC, _BM), lambda i: (0, i)),
            pl.BlockSpec((_BM, D_IN), lambda i: (i, 0)),
            pl.BlockSpec((D_IN, D_HID), lambda i: (0, 0)),
        ],
        out_specs=pl.BlockSpec((_BM, D_HID), lambda i: (i, 0)),
        out_shape=jax.ShapeDtypeStruct((NPAD, D_HID), jnp.float32),
    )(deg2, x, W1)

    s64 = _sc_agg64(g1, src3, dst3, zeros64)

    u = pl.pallas_call(
        _tc2_body,
        grid=(_GRID,),
        in_specs=[
            pl.BlockSpec((NC, _BM), lambda i: (0, i)),
            pl.BlockSpec((NC, _BM, D_HID), lambda i: (0, i, 0)),
            pl.BlockSpec((_BM, D_HID), lambda i: (i, 0)),
            pl.BlockSpec((D_HID, 1), lambda i: (0, 0)),
            pl.BlockSpec((1, D_HID), lambda i: (0, 0)),
        ],
        out_specs=pl.BlockSpec((_BF, 128), lambda i: (i, 0)),
        out_shape=jax.ShapeDtypeStruct((NF, 128), jnp.float32),
    )(deg2, s64, g1, W2, b1.reshape(1, D_HID))

    s2 = _sc_agg1(u.reshape(NPAD), src3, dst3, zeros1)

    outf = pl.pallas_call(
        _tc3_body,
        out_shape=jax.ShapeDtypeStruct((NF, 128), jnp.float32),
    )(deg2.reshape(NC, NF, 128), s2.reshape(NC, NF, 128), u, b2.reshape(1, 1))

    return outf.reshape(NPAD)[:N].reshape(N, 1)


# speedup vs baseline: 74.7720x; 1.0618x over previous
import functools

import jax
import jax.numpy as jnp
from jax import lax
from jax.experimental import pallas as pl
from jax.experimental.pallas import tpu as pltpu
from jax.experimental.pallas import tpu_sc as plsc

N = 10000
D_IN = 128
D_HID = 64
E = 320000
NC = 2
NS = 16
NW = NC * NS
EPW = E // NW
CH = 125
K = EPW // CH
NBUF = 4
NPAD = 10240
RT = NPAD // NS
NF = NPAD // 128
_BM = 2048
_GRID = NPAD // _BM
_BF = NF // _GRID

_mesh = plsc.VectorSubcoreMesh(core_axis_name="c", subcore_axis_name="s")
_sc_params = pltpu.CompilerParams(
    use_tc_tiling_on_sc=False, disable_bounds_checks=True)


@functools.partial(
    pl.kernel,
    out_type=jax.ShapeDtypeStruct((NC, NPAD), jnp.float32),
    mesh=_mesh,
    compiler_params=_sc_params,
    scratch_types=[
        pltpu.VMEM((K, CH), jnp.int32),
        pltpu.VMEM((128,), jnp.float32),
        pltpu.SemaphoreType.DMA,
        pltpu.SemaphoreType.DMA,
        pltpu.VMEM_SHARED((NPAD,), jnp.float32),
    ],
)
def _sc_degree(ei_hbm, ones_hbm, zeros_hbm, out_hbm, didx, ones_v, s0, s1, acc):
    cid = lax.axis_index("c")
    sid = lax.axis_index("s")
    wid = sid * NC + cid
    pltpu.sync_copy(ei_hbm.at[1, wid], didx)
    pltpu.sync_copy(ones_hbm, ones_v)
    pltpu.sync_copy(zeros_hbm.at[pl.ds(sid * RT, RT)], acc.at[pl.ds(sid * RT, RT)])
    plsc.subcore_barrier()

    ones_row = ones_v.at[pl.ds(0, CH)]

    def step(jj, carry):
        j0 = 2 * jj
        pltpu.async_copy(ones_row, acc.at[didx.at[j0]], s0, add=True)
        pltpu.async_copy(ones_row, acc.at[didx.at[j0 + 1]], s1, add=True)
        pltpu.make_async_copy(ones_row, acc.at[didx.at[j0]], s0).wait()
        pltpu.make_async_copy(ones_row, acc.at[didx.at[j0 + 1]], s1).wait()
        return carry

    lax.fori_loop(0, K // 2, step, 0)
    plsc.subcore_barrier()
    pltpu.sync_copy(acc.at[pl.ds(sid * RT, RT)], out_hbm.at[cid, pl.ds(sid * RT, RT)])


@functools.partial(
    pl.kernel,
    out_type=jax.ShapeDtypeStruct((NC, NPAD, D_HID), jnp.float32),
    mesh=_mesh,
    compiler_params=_sc_params,
    scratch_types=(
        [pltpu.VMEM((K, CH), jnp.int32)] * 2
        + [pltpu.VMEM((CH, D_HID), jnp.float32)] * NBUF
        + [pltpu.SemaphoreType.DMA] * (2 * NBUF)
        + [pltpu.VMEM_SHARED((NPAD, D_HID), jnp.float32)]
    ),
)
def _sc_agg64(g1_hbm, ei_hbm, zeros_hbm, out_hbm, sidx, didx, *rest):
    rows = rest[:NBUF]
    gsem = rest[NBUF:2 * NBUF]
    ssem = rest[2 * NBUF:3 * NBUF]
    acc = rest[3 * NBUF]
    cid = lax.axis_index("c")
    sid = lax.axis_index("s")
    wid = sid * NC + cid
    pltpu.sync_copy(ei_hbm.at[0, wid], sidx)
    pltpu.sync_copy(ei_hbm.at[1, wid], didx)
    pltpu.sync_copy(zeros_hbm.at[pl.ds(sid * RT, RT)], acc.at[pl.ds(sid * RT, RT)])
    plsc.subcore_barrier()

    for b in range(NBUF):
        pltpu.async_copy(g1_hbm.at[sidx.at[b]], rows[b], gsem[b])

    def step(jj, carry):
        for b in range(NBUF):
            j = NBUF * jj + b
            pltpu.make_async_copy(g1_hbm.at[sidx.at[j]], rows[b], gsem[b]).wait()
            pltpu.async_copy(rows[b], acc.at[didx.at[j]], ssem[b], add=True)

            @pl.when(j + NBUF < K)
            def _refill():
                pltpu.make_async_copy(rows[b], acc.at[didx.at[j]], ssem[b]).wait()
                pltpu.async_copy(g1_hbm.at[sidx.at[j + NBUF]], rows[b], gsem[b])

        return carry

    lax.fori_loop(0, K // NBUF, step, 0)
    for b in range(NBUF):
        j = K - NBUF + b
        pltpu.make_async_copy(rows[b], acc.at[didx.at[j]], ssem[b]).wait()
    plsc.subcore_barrier()
    pltpu.sync_copy(acc.at[pl.ds(sid * RT, RT)], out_hbm.at[cid, pl.ds(sid * RT, RT)])


@functools.partial(
    pl.kernel,
    out_type=jax.ShapeDtypeStruct((NC, NPAD), jnp.float32),
    mesh=_mesh,
    compiler_params=_sc_params,
    scratch_types=(
        [pltpu.VMEM((K, CH), jnp.int32)] * 2
        + [pltpu.VMEM((CH,), jnp.float32)] * NBUF
        + [pltpu.SemaphoreType.DMA] * (2 * NBUF)
        + [pltpu.VMEM_SHARED((NPAD,), jnp.float32),
           pltpu.VMEM_SHARED((NPAD,), jnp.float32)]
    ),
)
def _sc_agg1(u_hbm, ei_hbm, zeros_hbm, out_hbm, sidx, didx, *rest):
    vals = rest[:NBUF]
    gsem = rest[NBUF:2 * NBUF]
    ssem = rest[2 * NBUF:3 * NBUF]
    acc = rest[3 * NBUF]
    us = rest[3 * NBUF + 1]
    cid = lax.axis_index("c")
    sid = lax.axis_index("s")
    wid = sid * NC + cid
    pltpu.sync_copy(ei_hbm.at[0, wid], sidx)
    pltpu.sync_copy(ei_hbm.at[1, wid], didx)
    pltpu.sync_copy(zeros_hbm.at[pl.ds(sid * RT, RT)], acc.at[pl.ds(sid * RT, RT)])
    pltpu.sync_copy(u_hbm.at[pl.ds(sid * RT, RT)], us.at[pl.ds(sid * RT, RT)])
    plsc.subcore_barrier()

    for b in range(NBUF):
        pltpu.async_copy(us.at[sidx.at[b]], vals[b], gsem[b])

    def step(jj, carry):
        for b in range(NBUF):
            j = NBUF * jj + b
            pltpu.make_async_copy(us.at[sidx.at[j]], vals[b], gsem[b]).wait()
            pltpu.async_copy(vals[b], acc.at[didx.at[j]], ssem[b], add=True)

            @pl.when(j + NBUF < K)
            def _refill():
                pltpu.make_async_copy(vals[b], acc.at[didx.at[j]], ssem[b]).wait()
                pltpu.async_copy(us.at[sidx.at[j + NBUF]], vals[b], gsem[b])

        return carry

    lax.fori_loop(0, K // NBUF, step, 0)
    for b in range(NBUF):
        j = K - NBUF + b
        pltpu.make_async_copy(vals[b], acc.at[didx.at[j]], ssem[b]).wait()
    plsc.subcore_barrier()
    pltpu.sync_copy(acc.at[pl.ds(sid * RT, RT)], out_hbm.at[cid, pl.ds(sid * RT, RT)])


def _tc1_body(deg_ref, x_ref, w1_ref, g1_ref):
    dinv_c = lax.rsqrt(deg_ref[0] + deg_ref[1] + 1.0)[:, None]
    h = jnp.dot(x_ref[...], w1_ref[...], preferred_element_type=jnp.float32)
    g1_ref[...] = h * dinv_c


def _tc2_body(deg_ref, s_ref, g1_ref, w2_ref, b1_ref, u_ref):
    dinv_c = lax.rsqrt(deg_ref[0] + deg_ref[1] + 1.0)[:, None]
    s = s_ref[0] + s_ref[1] + g1_ref[...]
    h = jnp.maximum(s * dinv_c + b1_ref[...], 0.0)
    u_col = jnp.dot(h, w2_ref[...], preferred_element_type=jnp.float32) * dinv_c
    u_ref[...] = u_col.reshape(_BM).reshape(_BF, 128)


def _tc3_body(deg_ref, s2_ref, u_ref, b2_ref, o_ref):
    dinv = lax.rsqrt(deg_ref[0] + deg_ref[1] + 1.0)
    z = dinv * (s2_ref[0] + s2_ref[1] + u_ref[...]) + b2_ref[...]
    o_ref[...] = jax.nn.sigmoid(z)


def kernel(x, edge_index, W1, b1, W2, b2):
    ei4 = edge_index.astype(jnp.int32).reshape(2, NW, K, CH)
    zeros64 = jnp.zeros((NPAD, D_HID), jnp.float32)
    zeros1 = jnp.zeros((NPAD,), jnp.float32)
    ones128 = jnp.ones((128,), jnp.float32)

    deg2 = _sc_degree(ei4, ones128, zeros1)

    g1 = pl.pallas_call(
        _tc1_body,
        grid=(_GRID,),
        in_specs=[
            pl.BlockSpec((NC, _BM), lambda i: (0, i)),
            pl.BlockSpec((_BM, D_IN), lambda i: (i, 0)),
            pl.BlockSpec((D_IN, D_HID), lambda i: (0, 0)),
        ],
        out_specs=pl.BlockSpec((_BM, D_HID), lambda i: (i, 0)),
        out_shape=jax.ShapeDtypeStruct((NPAD, D_HID), jnp.float32),
    )(deg2, x, W1)

    s64 = _sc_agg64(g1, ei4, zeros64)

    u = pl.pallas_call(
        _tc2_body,
        grid=(_GRID,),
        in_specs=[
            pl.BlockSpec((NC, _BM), lambda i: (0, i)),
            pl.BlockSpec((NC, _BM, D_HID), lambda i: (0, i, 0)),
            pl.BlockSpec((_BM, D_HID), lambda i: (i, 0)),
            pl.BlockSpec((D_HID, 1), lambda i: (0, 0)),
            pl.BlockSpec((1, D_HID), lambda i: (0, 0)),
        ],
        out_specs=pl.BlockSpec((_BF, 128), lambda i: (i, 0)),
        out_shape=jax.ShapeDtypeStruct((NF, 128), jnp.float32),
    )(deg2, s64, g1, W2, b1.reshape(1, D_HID))

    s2 = _sc_agg1(u.reshape(NPAD), ei4, zeros1)

    outf = pl.pallas_call(
        _tc3_body,
        out_shape=jax.ShapeDtypeStruct((NF, 128), jnp.float32),
    )(deg2.reshape(NC, NF, 128), s2.reshape(NC, NF, 128), u, b2.reshape(1, 1))

    return outf.reshape(NPAD)[:N].reshape(N, 1)
